# baseline (device time: 16335 ns/iter reference)
import jax
import jax.numpy as jnp
from jax import lax
from jax.experimental import pallas as pl
from jax.experimental.pallas import tpu as pltpu

K = 8


def kernel(x):
    _, m, nh = x.shape
    rs = m // K

    def body(x_hbm, out_ref, xf, xb, rs_recv, cp_sems, send_sems, recv_sems):
        my_x = lax.axis_index("x")
        my_y = lax.axis_index("y")
        x_tgt = (1 - my_x, my_y)
        y_tgt = (my_x, 1 - my_y)

        cps = []
        for k in range(K):
            rows = pl.ds(k * rs, rs)
            cp = pltpu.make_async_copy(
                x_hbm.at[0, rows], xf.at[rows], cp_sems.at[k])
            cp.start()
            cps.append(cp)

        barrier_sem = pltpu.get_barrier_semaphore()
        for tgt in (x_tgt, y_tgt):
            pl.semaphore_signal(barrier_sem, inc=1, device_id=tgt,
                                device_id_type=pl.DeviceIdType.MESH)
        pl.semaphore_wait(barrier_sem, 2)

        raws = []
        for k in range(K):
            rows = pl.ds(k * rs, rs)
            cps[k].wait()
            xb[rows] = xf[rows].astype(jnp.bfloat16)
            r = pltpu.make_async_remote_copy(
                src_ref=xb.at[rows],
                dst_ref=rs_recv.at[rows],
                send_sem=send_sems.at[k], recv_sem=recv_sems.at[k],
                device_id=x_tgt, device_id_type=pl.DeviceIdType.MESH,
            )
            r.start()
            raws.append(r)

        own_cols = pl.ds(my_y * nh, nh)
        sums = []
        for k in range(K):
            rows = pl.ds(k * rs, rs)
            raws[k].wait_recv()
            out_ref[rows, own_cols] = xb[rows] + rs_recv[rows]
            s = pltpu.make_async_remote_copy(
                src_ref=out_ref.at[rows, own_cols],
                dst_ref=out_ref.at[rows, own_cols],
                send_sem=send_sems.at[K + k], recv_sem=recv_sems.at[K + k],
                device_id=y_tgt, device_id_type=pl.DeviceIdType.MESH,
            )
            s.start()
            sums.append(s)

        for s in sums:
            s.wait()
        for r in raws:
            r.wait_send()

    return pl.pallas_call(
        body,
        out_shape=jax.ShapeDtypeStruct((m, 2 * nh), jnp.bfloat16),
        in_specs=[pl.BlockSpec(memory_space=pl.ANY)],
        out_specs=pl.BlockSpec(memory_space=pltpu.VMEM),
        scratch_shapes=[
            pltpu.VMEM((m, nh), jnp.float32),
            pltpu.VMEM((m, nh), jnp.bfloat16),
            pltpu.VMEM((m, nh), jnp.bfloat16),
            pltpu.SemaphoreType.DMA((K,)),
            pltpu.SemaphoreType.DMA((2 * K,)),
            pltpu.SemaphoreType.DMA((2 * K,)),
        ],
        compiler_params=pltpu.CompilerParams(collective_id=0),
    )(x)


# device time: 14922 ns/iter; 1.0947x vs baseline; 1.0947x over previous
import jax
import jax.numpy as jnp
from jax import lax
from jax.experimental import pallas as pl
from jax.experimental.pallas import tpu as pltpu

K = 8


def kernel(x):
    _, m, nh = x.shape
    rs = m // K

    def body(x_ref, out_ref, xb, rs_recv, send_sems, recv_sems):
        my_x = lax.axis_index("x")
        my_y = lax.axis_index("y")
        x_tgt = (1 - my_x, my_y)
        y_tgt = (my_x, 1 - my_y)

        barrier_sem = pltpu.get_barrier_semaphore()
        for tgt in (x_tgt, y_tgt):
            pl.semaphore_signal(barrier_sem, inc=1, device_id=tgt,
                                device_id_type=pl.DeviceIdType.MESH)
        pl.semaphore_wait(barrier_sem, 2)

        raws = []
        for k in range(K):
            rows = pl.ds(k * rs, rs)
            xb[rows] = x_ref[0, rows].astype(jnp.bfloat16)
            r = pltpu.make_async_remote_copy(
                src_ref=xb.at[rows],
                dst_ref=rs_recv.at[rows],
                send_sem=send_sems.at[k], recv_sem=recv_sems.at[k],
                device_id=x_tgt, device_id_type=pl.DeviceIdType.MESH,
            )
            r.start()
            raws.append(r)

        own_cols = pl.ds(my_y * nh, nh)
        sums = []
        for k in range(K):
            rows = pl.ds(k * rs, rs)
            raws[k].wait_recv()
            out_ref[rows, own_cols] = xb[rows] + rs_recv[rows]
            s = pltpu.make_async_remote_copy(
                src_ref=out_ref.at[rows, own_cols],
                dst_ref=out_ref.at[rows, own_cols],
                send_sem=send_sems.at[K + k], recv_sem=recv_sems.at[K + k],
                device_id=y_tgt, device_id_type=pl.DeviceIdType.MESH,
            )
            s.start()
            sums.append(s)

        for s in sums:
            s.wait()
        for r in raws:
            r.wait_send()

    return pl.pallas_call(
        body,
        out_shape=jax.ShapeDtypeStruct((m, 2 * nh), jnp.bfloat16),
        in_specs=[pl.BlockSpec(memory_space=pltpu.VMEM)],
        out_specs=pl.BlockSpec(memory_space=pltpu.VMEM),
        scratch_shapes=[
            pltpu.VMEM((m, nh), jnp.bfloat16),
            pltpu.VMEM((m, nh), jnp.bfloat16),
            pltpu.SemaphoreType.DMA((2 * K,)),
            pltpu.SemaphoreType.DMA((2 * K,)),
        ],
        compiler_params=pltpu.CompilerParams(collective_id=0),
    )(x)
